# baseline (speedup 1.0000x reference)
import jax
import jax.numpy as jnp
from jax.experimental import pallas as pl
from jax.experimental.pallas import tpu as pltpu

_IMAGE_W = 1333
_IMAGE_H = 800
_MIN_SCORE = 0.05
_NMS_T = 0.5
_MAX_DET = 100
_N = 20000
_C = 80
_LANES = 128
_ROWS = 160
_NPAD = _ROWS * _LANES
_NEG = -1e30


def _decode_nms_kernel(cls_ref, reg_ref, anc_ref, out_ref, pk_ref, sc_ref):

    nblk = _N // _LANES
    eye_full = jnp.eye(_LANES, _LANES, dtype=jnp.float32)
    eye_tail = jnp.eye(_N - nblk * _LANES, _LANES, dtype=jnp.float32)
    ki = jax.lax.broadcasted_iota(jnp.int32, (_C, _LANES), 0)
    for i in range(nblk + 1):
        if i < nblk:
            blk = cls_ref[pl.ds(i * _LANES, _LANES), :]
            eye = eye_full
        else:
            blk = cls_ref[pl.ds(nblk * _LANES, _N - nblk * _LANES), :]
            eye = eye_tail
        t = jax.lax.dot_general(
            blk, eye, (((0,), (0,)), ((), ())),
            precision=jax.lax.Precision.HIGHEST,
            preferred_element_type=jnp.float32)
        m = jnp.max(t, axis=0, keepdims=True)
        karg = jnp.min(jnp.where(t == m, ki, _C), axis=0, keepdims=True)
        sc_ref[0, pl.ds(i, 1), :] = m
        sc_ref[1, pl.ds(i, 1), :] = karg.astype(jnp.float32)
    sc_ref[0, pl.ds(nblk + 1, _ROWS - nblk - 1), :] = jnp.zeros(
        (_ROWS - nblk - 1, _LANES), jnp.float32)

    scores = sc_ref[0]
    clsf = sc_ref[1]

    ax1 = anc_ref[0]
    ay1 = anc_ref[1]
    ax2 = anc_ref[2]
    ay2 = anc_ref[3]
    aw = ax2 - ax1
    ah = ay2 - ay1
    acx = ax1 + 0.5 * aw
    acy = ay1 + 0.5 * ah
    tx = reg_ref[0] * 0.1
    ty = reg_ref[1] * 0.1
    tw = reg_ref[2] * 0.2
    th = reg_ref[3] * 0.2
    pw = jnp.exp(tw) * aw
    ph = jnp.exp(th) * ah
    pcx = tx * aw + acx
    pcy = ty * ah + acy
    x1 = jnp.maximum(jnp.trunc(pcx - 0.5 * pw), 0.0)
    y1 = jnp.maximum(jnp.trunc(pcy - 0.5 * ph), 0.0)
    x2 = jnp.minimum(jnp.trunc(pcx + 0.5 * pw), float(_IMAGE_W - 1))
    y2 = jnp.minimum(jnp.trunc(pcy + 0.5 * ph), float(_IMAGE_H - 1))
    area = jnp.maximum(x2 - x1, 0.0) * jnp.maximum(y2 - y1, 0.0)

    pk_ref[0] = x1
    pk_ref[1] = y1
    pk_ref[2] = x2
    pk_ref[3] = y2
    pk_ref[4] = area
    pk_ref[5] = clsf

    rowi = jax.lax.broadcasted_iota(jnp.int32, (_ROWS, _LANES), 0)
    lanei = jax.lax.broadcasted_iota(jnp.int32, (_ROWS, _LANES), 1)
    linear = rowi * _LANES + lanei
    valid = (scores > _MIN_SCORE) & (linear < _N)
    msco = jnp.where(valid, scores, _NEG)

    li = jax.lax.broadcasted_iota(jnp.int32, (1, _LANES), 1)

    def step(t, msco):
        sbest = jnp.max(msco)
        has = sbest > -1e29
        sel = msco == sbest
        idx = jnp.min(jnp.where(sel, linear, _NPAD))
        sel1 = linear == idx

        r = idx // _LANES
        lsel = li == (idx - r * _LANES)

        def ext(j):
            v = pk_ref[j, pl.ds(r, 1), :]
            return jnp.sum(jnp.where(lsel, v, 0.0), axis=1, keepdims=True)

        rx1 = ext(0)
        ry1 = ext(1)
        rx2 = ext(2)
        ry2 = ext(3)
        rar = ext(4)
        rc = ext(5)

        ix1 = jnp.maximum(rx1, x1)
        iy1 = jnp.maximum(ry1, y1)
        ix2 = jnp.minimum(rx2, x2)
        iy2 = jnp.minimum(ry2, y2)
        inter = jnp.maximum(ix2 - ix1, 0.0) * jnp.maximum(iy2 - iy1, 0.0)
        supp = ((3.0 * inter > rar + area) & (clsf == rc)) | sel1
        msco = jnp.where(has & supp, _NEG, msco)

        row = jnp.where(
            has,
            jnp.where(
                li == 0, rx1,
                jnp.where(
                    li == 1, ry1,
                    jnp.where(
                        li == 2, rx2,
                        jnp.where(
                            li == 3, ry2,
                            jnp.where(li == 4, sbest,
                                      jnp.where(li == 5, rc, 0.0)))))),
            jnp.where(li == 5, -1.0, 0.0),
        )
        out_ref[pl.ds(t, 1), :] = row
        return msco

    jax.lax.fori_loop(0, _MAX_DET, step, msco)


def kernel(cls_heads, reg_heads, batch_anchors):
    cls = cls_heads[0]
    reg = reg_heads[0]
    anc = batch_anchors[0]

    pad = _NPAD - _N
    reg_t = jnp.pad(reg, ((0, pad), (0, 0))).T.reshape(4, _ROWS, _LANES)
    anc_t = jnp.pad(anc, ((0, pad), (0, 0))).T.reshape(4, _ROWS, _LANES)

    out = pl.pallas_call(
        _decode_nms_kernel,
        out_shape=jax.ShapeDtypeStruct((_MAX_DET + 4, _LANES), jnp.float32),
        scratch_shapes=[pltpu.VMEM((6, _ROWS, _LANES), jnp.float32),
                        pltpu.VMEM((2, _ROWS, _LANES), jnp.float32)],
    )(cls, reg_t, anc_t)

    detections = out[:_MAX_DET, :5][None]
    classes = out[:_MAX_DET, 5].astype(jnp.int32)[None]
    return detections, classes


# speedup vs baseline: 1.1859x; 1.1859x over previous
import jax
import jax.numpy as jnp
from jax.experimental import pallas as pl
from jax.experimental.pallas import tpu as pltpu

_IMAGE_W = 1333
_IMAGE_H = 800
_MIN_SCORE = 0.05
_NMS_T = 0.5
_MAX_DET = 100
_N = 20000
_C = 80
_LANES = 128
_ROWS = 160
_NPAD = _ROWS * _LANES
_NEG = -1e30


def _decode_nms_kernel(cls_ref, reg_ref, anc_ref, out_ref, pk_ref):

    best = cls_ref[0]
    bestk = jnp.zeros((_ROWS, _LANES), jnp.float32)
    for k in range(1, _C):
        xk = cls_ref[k]
        m = xk > best
        best = jnp.where(m, xk, best)
        bestk = jnp.where(m, float(k), bestk)
    scores = best
    clsf = bestk

    ax1 = anc_ref[0]
    ay1 = anc_ref[1]
    ax2 = anc_ref[2]
    ay2 = anc_ref[3]
    aw = ax2 - ax1
    ah = ay2 - ay1
    acx = ax1 + 0.5 * aw
    acy = ay1 + 0.5 * ah
    tx = reg_ref[0] * 0.1
    ty = reg_ref[1] * 0.1
    tw = reg_ref[2] * 0.2
    th = reg_ref[3] * 0.2
    pw = jnp.exp(tw) * aw
    ph = jnp.exp(th) * ah
    pcx = tx * aw + acx
    pcy = ty * ah + acy
    x1 = jnp.maximum(jnp.trunc(pcx - 0.5 * pw), 0.0)
    y1 = jnp.maximum(jnp.trunc(pcy - 0.5 * ph), 0.0)
    x2 = jnp.minimum(jnp.trunc(pcx + 0.5 * pw), float(_IMAGE_W - 1))
    y2 = jnp.minimum(jnp.trunc(pcy + 0.5 * ph), float(_IMAGE_H - 1))
    area = jnp.maximum(x2 - x1, 0.0) * jnp.maximum(y2 - y1, 0.0)

    pk_ref[0] = x1
    pk_ref[1] = y1
    pk_ref[2] = x2
    pk_ref[3] = y2
    pk_ref[4] = area
    pk_ref[5] = clsf

    rowi = jax.lax.broadcasted_iota(jnp.int32, (_ROWS, _LANES), 0)
    lanei = jax.lax.broadcasted_iota(jnp.int32, (_ROWS, _LANES), 1)
    linear = rowi * _LANES + lanei
    valid = (scores > _MIN_SCORE) & (linear < _N)
    msco = jnp.where(valid, scores, _NEG)

    li = jax.lax.broadcasted_iota(jnp.int32, (1, _LANES), 1)

    def step(t, msco):
        sbest = jnp.max(msco)
        has = sbest > -1e29
        sel = msco == sbest
        idx = jnp.min(jnp.where(sel, linear, _NPAD))
        sel1 = linear == idx

        r = idx // _LANES
        lsel = li == (idx - r * _LANES)

        def ext(j):
            v = pk_ref[j, pl.ds(r, 1), :]
            return jnp.sum(jnp.where(lsel, v, 0.0), axis=1, keepdims=True)

        rx1 = ext(0)
        ry1 = ext(1)
        rx2 = ext(2)
        ry2 = ext(3)
        rar = ext(4)
        rc = ext(5)

        ix1 = jnp.maximum(rx1, x1)
        iy1 = jnp.maximum(ry1, y1)
        ix2 = jnp.minimum(rx2, x2)
        iy2 = jnp.minimum(ry2, y2)
        inter = jnp.maximum(ix2 - ix1, 0.0) * jnp.maximum(iy2 - iy1, 0.0)
        supp = ((3.0 * inter > rar + area) & (clsf == rc)) | sel1
        msco = jnp.where(has & supp, _NEG, msco)

        row = jnp.where(
            has,
            jnp.where(
                li == 0, rx1,
                jnp.where(
                    li == 1, ry1,
                    jnp.where(
                        li == 2, rx2,
                        jnp.where(
                            li == 3, ry2,
                            jnp.where(li == 4, sbest,
                                      jnp.where(li == 5, rc, 0.0)))))),
            jnp.where(li == 5, -1.0, 0.0),
        )
        out_ref[pl.ds(t, 1), :] = row
        return msco

    jax.lax.fori_loop(0, _MAX_DET, step, msco)


def kernel(cls_heads, reg_heads, batch_anchors):
    cls = cls_heads[0]
    reg = reg_heads[0]
    anc = batch_anchors[0]

    pad = _NPAD - _N
    cls3 = jnp.pad(cls, ((0, pad), (0, 0))).reshape(_ROWS, _LANES, _C)
    cls_t = jnp.einsum('rlc,lm->crm', cls3, jnp.eye(_LANES, dtype=cls.dtype),
                       precision=jax.lax.Precision.HIGHEST)
    reg_t = jnp.pad(reg, ((0, pad), (0, 0))).T.reshape(4, _ROWS, _LANES)
    anc_t = jnp.pad(anc, ((0, pad), (0, 0))).T.reshape(4, _ROWS, _LANES)

    out = pl.pallas_call(
        _decode_nms_kernel,
        out_shape=jax.ShapeDtypeStruct((_MAX_DET + 4, _LANES), jnp.float32),
        scratch_shapes=[pltpu.VMEM((6, _ROWS, _LANES), jnp.float32)],
    )(cls_t, reg_t, anc_t)

    detections = out[:_MAX_DET, :5][None]
    classes = out[:_MAX_DET, 5].astype(jnp.int32)[None]
    return detections, classes


# speedup vs baseline: 1.2836x; 1.0824x over previous
import jax
import jax.numpy as jnp
from jax.experimental import pallas as pl
from jax.experimental.pallas import tpu as pltpu

_IMAGE_W = 1333
_IMAGE_H = 800
_MIN_SCORE = 0.05
_NMS_T = 0.5
_MAX_DET = 100
_N = 20000
_C = 80
_LANES = 128
_ROWS = 160
_NPAD = _ROWS * _LANES
_NEG = -1e30


def _decode_nms_kernel(cls_ref, reg_ref, anc_ref, out_ref, pk_ref):

    best = cls_ref[0]
    bestk = jnp.zeros((_ROWS, _LANES), jnp.float32)
    for k in range(1, _C):
        xk = cls_ref[k]
        m = xk > best
        best = jnp.where(m, xk, best)
        bestk = jnp.where(m, float(k), bestk)
    scores = best
    clsf = bestk

    ax1 = anc_ref[0]
    ay1 = anc_ref[1]
    ax2 = anc_ref[2]
    ay2 = anc_ref[3]
    aw = ax2 - ax1
    ah = ay2 - ay1
    acx = ax1 + 0.5 * aw
    acy = ay1 + 0.5 * ah
    tx = reg_ref[0] * 0.1
    ty = reg_ref[1] * 0.1
    tw = reg_ref[2] * 0.2
    th = reg_ref[3] * 0.2
    pw = jnp.exp(tw) * aw
    ph = jnp.exp(th) * ah
    pcx = tx * aw + acx
    pcy = ty * ah + acy
    x1 = jnp.maximum(jnp.trunc(pcx - 0.5 * pw), 0.0)
    y1 = jnp.maximum(jnp.trunc(pcy - 0.5 * ph), 0.0)
    x2 = jnp.minimum(jnp.trunc(pcx + 0.5 * pw), float(_IMAGE_W - 1))
    y2 = jnp.minimum(jnp.trunc(pcy + 0.5 * ph), float(_IMAGE_H - 1))
    area = jnp.maximum(x2 - x1, 0.0) * jnp.maximum(y2 - y1, 0.0)

    pk_ref[0] = x1
    pk_ref[1] = y1
    pk_ref[2] = x2
    pk_ref[3] = y2
    pk_ref[4] = area
    pk_ref[5] = clsf

    rowi = jax.lax.broadcasted_iota(jnp.int32, (_ROWS, _LANES), 0)
    lanei = jax.lax.broadcasted_iota(jnp.int32, (_ROWS, _LANES), 1)
    linear = rowi * _LANES + lanei
    valid = (scores > _MIN_SCORE) & (linear < _N)
    msco = jnp.where(valid, scores, _NEG)

    li = jax.lax.broadcasted_iota(jnp.int32, (1, _LANES), 1)

    def step(t, msco):
        sbest = jnp.max(msco)
        has = sbest > -1e29
        sel = msco == sbest
        idx = jnp.min(jnp.where(sel, linear, _NPAD))
        sel1 = linear == idx

        r = idx // _LANES
        lsel = li == (idx - r * _LANES)

        def ext(j):
            v = pk_ref[j, pl.ds(r, 1), :]
            return jnp.sum(jnp.where(lsel, v, 0.0), axis=1, keepdims=True)

        rx1 = ext(0)
        ry1 = ext(1)
        rx2 = ext(2)
        ry2 = ext(3)
        rar = ext(4)
        rc = ext(5)

        ix1 = jnp.maximum(rx1, x1)
        iy1 = jnp.maximum(ry1, y1)
        ix2 = jnp.minimum(rx2, x2)
        iy2 = jnp.minimum(ry2, y2)
        inter = jnp.maximum(ix2 - ix1, 0.0) * jnp.maximum(iy2 - iy1, 0.0)
        supp = ((3.0 * inter > rar + area) & (clsf == rc)) | sel1
        msco = jnp.where(has & supp, _NEG, msco)

        row = jnp.where(
            has,
            jnp.where(
                li == 0, rx1,
                jnp.where(
                    li == 1, ry1,
                    jnp.where(
                        li == 2, rx2,
                        jnp.where(
                            li == 3, ry2,
                            jnp.where(li == 4, sbest,
                                      jnp.where(li == 5, rc, 0.0)))))),
            jnp.where(li == 5, -1.0, 0.0),
        )
        out_ref[pl.ds(t, 1), :] = row
        return msco

    jax.lax.fori_loop(0, _MAX_DET, step, msco)


def kernel(cls_heads, reg_heads, batch_anchors):
    cls = cls_heads[0]
    reg = reg_heads[0]
    anc = batch_anchors[0]

    pad = _NPAD - _N
    cls_t = jnp.pad(cls.T, ((0, 0), (0, pad))).reshape(_C, _ROWS, _LANES)
    reg_t = jnp.pad(reg, ((0, pad), (0, 0))).T.reshape(4, _ROWS, _LANES)
    anc_t = jnp.pad(anc, ((0, pad), (0, 0))).T.reshape(4, _ROWS, _LANES)

    out = pl.pallas_call(
        _decode_nms_kernel,
        out_shape=jax.ShapeDtypeStruct((_MAX_DET + 4, _LANES), jnp.float32),
        scratch_shapes=[pltpu.VMEM((6, _ROWS, _LANES), jnp.float32)],
    )(cls_t, reg_t, anc_t)

    detections = out[:_MAX_DET, :5][None]
    classes = out[:_MAX_DET, 5].astype(jnp.int32)[None]
    return detections, classes


# speedup vs baseline: 1.3300x; 1.0362x over previous
import jax
import jax.numpy as jnp
from jax.experimental import pallas as pl
from jax.experimental.pallas import tpu as pltpu

_IMAGE_W = 1333
_IMAGE_H = 800
_MIN_SCORE = 0.05
_NMS_T = 0.5
_MAX_DET = 100
_N = 20000
_C = 80
_LANES = 128
_ROWS = 160
_NG = _ROWS // 8
_NPAD = _ROWS * _LANES
_NEG = -1e30


def _decode_nms_kernel(cls_ref, reg_ref, anc_ref, out_ref, pk_ref):

    best = cls_ref[0]
    bestk = jnp.zeros((_ROWS, _LANES), jnp.float32)
    for k in range(1, _C):
        xk = cls_ref[k]
        m = xk > best
        best = jnp.where(m, xk, best)
        bestk = jnp.where(m, float(k), bestk)
    scores = best
    clsf = bestk

    ax1 = anc_ref[0]
    ay1 = anc_ref[1]
    ax2 = anc_ref[2]
    ay2 = anc_ref[3]
    aw = ax2 - ax1
    ah = ay2 - ay1
    acx = ax1 + 0.5 * aw
    acy = ay1 + 0.5 * ah
    tx = reg_ref[0] * 0.1
    ty = reg_ref[1] * 0.1
    tw = reg_ref[2] * 0.2
    th = reg_ref[3] * 0.2
    pw = jnp.exp(tw) * aw
    ph = jnp.exp(th) * ah
    pcx = tx * aw + acx
    pcy = ty * ah + acy
    x1 = jnp.maximum(jnp.trunc(pcx - 0.5 * pw), 0.0)
    y1 = jnp.maximum(jnp.trunc(pcy - 0.5 * ph), 0.0)
    x2 = jnp.minimum(jnp.trunc(pcx + 0.5 * pw), float(_IMAGE_W - 1))
    y2 = jnp.minimum(jnp.trunc(pcy + 0.5 * ph), float(_IMAGE_H - 1))
    area = jnp.maximum(x2 - x1, 0.0) * jnp.maximum(y2 - y1, 0.0)

    pk_ref[0] = x1
    pk_ref[1] = y1
    pk_ref[2] = x2
    pk_ref[3] = y2
    pk_ref[4] = clsf

    rowi = jax.lax.broadcasted_iota(jnp.int32, (_ROWS, _LANES), 0)
    lanei = jax.lax.broadcasted_iota(jnp.int32, (_ROWS, _LANES), 1)
    linear = rowi * _LANES + lanei
    valid = (scores > _MIN_SCORE) & (linear < _N)
    msco = jnp.where(valid, scores, _NEG)

    li = jax.lax.broadcasted_iota(jnp.int32, (1, _LANES), 1)
    si8 = jax.lax.broadcasted_iota(jnp.int32, (8, _LANES), 0)
    li8 = jax.lax.broadcasted_iota(jnp.int32, (8, _LANES), 1)

    def fold_w(m3):
        w = m3[0]
        wr = jnp.zeros((8, _LANES), jnp.int32)
        for r in range(1, _NG):
            c = m3[r] > w
            w = jnp.where(c, m3[r], w)
            wr = jnp.where(c, r, wr)
        return w, wr

    w0, wr0 = fold_w(msco.reshape(_NG, 8, _LANES))

    def step(t, carry):
        msco, w, wr = carry
        sbest = jnp.max(w)
        has = sbest > -1e29
        cand = jnp.where(w == sbest, (wr * 8 + si8) * _LANES + li8, _NPAD)
        idx = jnp.min(cand)

        r = idx // _LANES
        lsel = li == (idx - r * _LANES)

        def ext(j):
            v = pk_ref[j, pl.ds(r, 1), :]
            return jnp.sum(jnp.where(lsel, v, 0.0), axis=1, keepdims=True)

        rx1 = ext(0)
        ry1 = ext(1)
        rx2 = ext(2)
        ry2 = ext(3)
        rc = ext(4)
        rar = jnp.maximum(rx2 - rx1, 0.0) * jnp.maximum(ry2 - ry1, 0.0)

        ix1 = jnp.maximum(rx1, x1)
        iy1 = jnp.maximum(ry1, y1)
        ix2 = jnp.minimum(rx2, x2)
        iy2 = jnp.minimum(ry2, y2)
        inter = jnp.maximum(ix2 - ix1, 0.0) * jnp.maximum(iy2 - iy1, 0.0)
        supp = ((3.0 * inter > rar + area) & (clsf == rc)) | (linear == idx)
        msco = jnp.where(supp, _NEG, msco)
        w, wr = fold_w(msco.reshape(_NG, 8, _LANES))

        row = jnp.where(
            has,
            jnp.where(
                li == 0, rx1,
                jnp.where(
                    li == 1, ry1,
                    jnp.where(
                        li == 2, rx2,
                        jnp.where(
                            li == 3, ry2,
                            jnp.where(li == 4, sbest,
                                      jnp.where(li == 5, rc, 0.0)))))),
            jnp.where(li == 5, -1.0, 0.0),
        )
        out_ref[pl.ds(t, 1), :] = row
        return msco, w, wr

    jax.lax.fori_loop(0, _MAX_DET, step, (msco, w0, wr0))


def kernel(cls_heads, reg_heads, batch_anchors):
    cls = cls_heads[0]
    reg = reg_heads[0]
    anc = batch_anchors[0]

    pad = _NPAD - _N
    cls_t = jnp.pad(cls.T, ((0, 0), (0, pad))).reshape(_C, _ROWS, _LANES)
    reg_t = jnp.pad(reg.T, ((0, 0), (0, pad))).reshape(4, _ROWS, _LANES)
    anc_t = jnp.pad(anc.T, ((0, 0), (0, pad))).reshape(4, _ROWS, _LANES)

    out = pl.pallas_call(
        _decode_nms_kernel,
        out_shape=jax.ShapeDtypeStruct((_MAX_DET + 4, _LANES), jnp.float32),
        scratch_shapes=[pltpu.VMEM((5, _ROWS, _LANES), jnp.float32)],
    )(cls_t, reg_t, anc_t)

    detections = out[:_MAX_DET, :5][None]
    classes = out[:_MAX_DET, 5].astype(jnp.int32)[None]
    return detections, classes
